# baseline (speedup 1.0000x reference)
import functools
import math

import jax
import jax.numpy as jnp
from jax.experimental import pallas as pl
from jax.experimental.pallas import tpu as pltpu

B = 4
T = 2048
D_IN = 768
D_HID = 96
N_EXPERTS = 8
GP = 128
G4 = 4 * GP


def _pad_gate_cols(Wt):
    z = jnp.zeros((Wt.shape[0], GP - D_HID), Wt.dtype)
    parts = []
    for j in range(4):
        parts.append(Wt[:, D_HID * j:D_HID * (j + 1)])
        parts.append(z)
    return jnp.concatenate(parts, axis=1)


def _lstm_attn_moe_kernel(nc, tc,
                          x_ref, maskt_ref, len_ref,
                          wih_ref, whh_ref, b_ref,
                          wq_ref, bq_ref, wk_ref, bk_ref, wv_ref, bv_ref,
                          wg_ref, bg_ref, we1_ref, be1_ref,
                          we2_ref, sseg_ref, be2_ref,
                          yhat_ref, alpha_ref, gl_ref,
                          H_scr, h_scr, c_scr, xp_scr):
    k = pl.program_id(0)

    @pl.when(k == 0)
    def _init():
        h_scr[...] = jnp.zeros_like(h_scr)
        c_scr[...] = jnp.zeros_like(c_scr)

    xb = x_ref[...]
    x2 = xb.reshape(tc * B, D_IN)
    xp = jnp.dot(x2, wih_ref[...], preferred_element_type=jnp.float32)
    xp_scr[...] = (xp + b_ref[...]).reshape(tc // 2, 2 * B, G4)

    whh = whh_ref[...]

    def substep(xpt, h, c):
        hb = h.astype(jnp.bfloat16)
        gates = xpt + jnp.dot(hb, whh, preferred_element_type=jnp.float32)
        ig = jax.nn.sigmoid(gates[:, 0:GP])
        fg = jax.nn.sigmoid(gates[:, GP:2 * GP])
        gg = jnp.tanh(gates[:, 2 * GP:3 * GP])
        og = jax.nn.sigmoid(gates[:, 3 * GP:4 * GP])
        c_new = fg * c + ig * gg
        h_new = og * jnp.tanh(c_new)
        return h_new, c_new

    def step(j, carry):
        h, c = carry
        slab = xp_scr[j]
        h1, c1 = substep(slab[0:B], h, c)
        h2, c2 = substep(slab[B:2 * B], h1, c1)
        H_scr[k * (tc // 2) + j] = jnp.concatenate([h1, h2], axis=0)
        return (h2, c2)

    hF, cF = jax.lax.fori_loop(0, tc // 2, step, (h_scr[...], c_scr[...]),
                               unroll=8)
    h_scr[...] = hF
    c_scr[...] = cF

    @pl.when(k == nc - 1)
    def _final():
        sub_iota = jax.lax.broadcasted_iota(jnp.int32, (2 * B, 1), 0)
        rows = []
        for b in range(B):
            r = B * (len_ref[b] - 1) + b
            slab = H_scr[r // (2 * B)]
            onehot = (sub_iota == r % (2 * B)).astype(jnp.float32)
            rows.append(jnp.sum(slab * onehot, axis=0, keepdims=True))
        hT = jnp.concatenate(rows, axis=0)
        H = H_scr[...].reshape(T * B, GP)
        K2 = jnp.dot(H, wk_ref[...], preferred_element_type=jnp.float32) + bk_ref[...]
        V2 = jnp.dot(H, wv_ref[...], preferred_element_type=jnp.float32) + bv_ref[...]
        K4 = K2.reshape(T, B, D_HID)
        V4 = V2.reshape(T, B, D_HID)
        Q = jnp.dot(hT, wq_ref[...], preferred_element_type=jnp.float32) + bq_ref[...]
        L = jnp.sum(Q[None, :, :] * K4, axis=-1) * (1.0 / math.sqrt(D_HID))
        mf = maskt_ref[...]
        Lm = mf * L + (1.0 - mf) * (-1e9)
        mx = jnp.max(Lm, axis=0, keepdims=True)
        e = jnp.exp(Lm - mx)
        alpha = e / jnp.sum(e, axis=0, keepdims=True)
        alpha_ref[...] = alpha.T
        ctx = jnp.sum(alpha[:, :, None] * V4, axis=0)
        hT96 = hT[:, :D_HID]
        feats = jnp.concatenate([ctx, hT96], axis=-1)
        gl = jnp.dot(feats, wg_ref[...], preferred_element_type=jnp.float32) + bg_ref[...]
        gl_ref[...] = gl
        h1 = jnp.maximum(
            jnp.dot(feats, we1_ref[...], preferred_element_type=jnp.float32) + be1_ref[...],
            0.0)
        outs = jnp.dot(h1 * we2_ref[...], sseg_ref[...],
                       preferred_element_type=jnp.float32) + be2_ref[...]
        iota = jax.lax.broadcasted_iota(jnp.int32, (B, N_EXPERTS), 1)
        v1 = jnp.max(gl, axis=-1, keepdims=True)
        i1 = jnp.min(jnp.where(gl == v1, iota, N_EXPERTS), axis=-1, keepdims=True)
        m1 = iota == i1
        gl2 = jnp.where(m1, -1e30, gl)
        v2 = jnp.max(gl2, axis=-1, keepdims=True)
        i2 = jnp.min(jnp.where(gl2 == v2, iota, N_EXPERTS), axis=-1, keepdims=True)
        m2 = iota == i2
        d = jnp.exp(v2 - v1)
        p1 = 1.0 / (1.0 + d)
        p2 = d / (1.0 + d)
        s1 = jnp.sum(jnp.where(m1, outs, 0.0), axis=-1, keepdims=True)
        s2 = jnp.sum(jnp.where(m2, outs, 0.0), axis=-1, keepdims=True)
        yhat_ref[...] = p1 * s1 + p2 * s2


TC = 256


@jax.jit
def _run(x_tm, maskt, len2, wihT, whhT, b2,
         wqT, bq2, wkT, bk2, wvT, bv2,
         wgT, bg2, we1T, be1r, we2f, sseg, be2r):
    tc = TC
    nc = T // tc
    full = lambda shape: pl.BlockSpec(shape, lambda k: tuple(0 for _ in shape))
    out_shapes = (
        jax.ShapeDtypeStruct((B, 1), jnp.float32),
        jax.ShapeDtypeStruct((B, T), jnp.float32),
        jax.ShapeDtypeStruct((B, N_EXPERTS), jnp.float32),
    )
    return pl.pallas_call(
        functools.partial(_lstm_attn_moe_kernel, nc, tc),
        grid=(nc,),
        in_specs=[
            pl.BlockSpec((tc, B, D_IN), lambda k: (k, 0, 0)),
            full((T, B)),
            pl.BlockSpec(memory_space=pltpu.SMEM),
            full((D_IN, G4)),
            full((GP, G4)),
            full((1, G4)),
            full((GP, D_HID)), full((1, D_HID)),
            full((GP, D_HID)), full((1, D_HID)),
            full((GP, D_HID)), full((1, D_HID)),
            full((2 * D_HID, N_EXPERTS)), full((1, N_EXPERTS)),
            full((2 * D_HID, N_EXPERTS * D_HID)), full((1, N_EXPERTS * D_HID)),
            full((1, N_EXPERTS * D_HID)),
            full((N_EXPERTS * D_HID, N_EXPERTS)),
            full((1, N_EXPERTS)),
        ],
        out_specs=(full((B, 1)), full((B, T)), full((B, N_EXPERTS))),
        out_shape=out_shapes,
        scratch_shapes=[
            pltpu.VMEM((T // 2, 2 * B, GP), jnp.float32),
            pltpu.VMEM((B, GP), jnp.float32),
            pltpu.VMEM((B, GP), jnp.float32),
            pltpu.VMEM((tc // 2, 2 * B, G4), jnp.float32),
        ],
        compiler_params=pltpu.CompilerParams(
            dimension_semantics=("arbitrary",)),
    )(x_tm, maskt, len2, wihT, whhT, b2,
      wqT, bq2, wkT, bk2, wvT, bv2,
      wgT, bg2, we1T, be1r, we2f, sseg, be2r)


def _pad_rows(W):
    return jnp.pad(W, ((0, GP - D_HID), (0, 0)))


def kernel(x, lengths, mask, W_ih, W_hh, b_ih, b_hh, Wq, bq, Wk, bk, Wv, bv,
           Wg, bg, We1, be1, We2, be2):
    x_tm = jnp.swapaxes(x, 0, 1)
    maskt = mask.astype(jnp.float32).T
    len2 = lengths.astype(jnp.int32)
    wihT = _pad_gate_cols(W_ih.T)
    whhT = _pad_rows(_pad_gate_cols(W_hh.T)).astype(jnp.bfloat16)
    b2 = _pad_gate_cols((b_ih + b_hh).reshape(1, -1))
    we1T = We1.reshape(N_EXPERTS * D_HID, 2 * D_HID).T
    be1r = be1.reshape(1, N_EXPERTS * D_HID)
    we2f = We2.reshape(1, N_EXPERTS * D_HID)
    sseg = jnp.repeat(jnp.eye(N_EXPERTS, dtype=jnp.float32), D_HID, axis=0)
    be2r = be2.reshape(1, N_EXPERTS)
    return _run(x_tm, maskt, len2, wihT, whhT, b2,
                _pad_rows(Wq.T), bq.reshape(1, -1),
                _pad_rows(Wk.T), bk.reshape(1, -1),
                _pad_rows(Wv.T), bv.reshape(1, -1),
                Wg.T, bg.reshape(1, -1), we1T, be1r, we2f, sseg, be2r)


# speedup vs baseline: 1.0072x; 1.0072x over previous
import functools
import math

import jax
import jax.numpy as jnp
from jax.experimental import pallas as pl
from jax.experimental.pallas import tpu as pltpu

B = 4
T = 2048
D_IN = 768
D_HID = 96
N_EXPERTS = 8
GP = 128
G4 = 4 * GP


def _pad_gate_cols(Wt):
    z = jnp.zeros((Wt.shape[0], GP - D_HID), Wt.dtype)
    parts = []
    for j in range(4):
        parts.append(Wt[:, D_HID * j:D_HID * (j + 1)])
        parts.append(z)
    return jnp.concatenate(parts, axis=1)


def _lstm_attn_moe_kernel(nc, tc,
                          x_ref, maskt_ref, len_ref,
                          wih_ref, whh_ref, b_ref,
                          wq_ref, bq_ref, wk_ref, bk_ref, wv_ref, bv_ref,
                          wg_ref, bg_ref, we1_ref, be1_ref,
                          we2_ref, sseg_ref, be2_ref,
                          yhat_ref, alpha_ref, gl_ref,
                          H_scr, h_scr, c_scr, xp_scr):
    k = pl.program_id(0)

    @pl.when(k == 0)
    def _init():
        h_scr[...] = jnp.zeros_like(h_scr)
        c_scr[...] = jnp.zeros_like(c_scr)

    xb = x_ref[...]
    x2 = xb.reshape(tc * B, D_IN)
    xp = jnp.dot(x2, wih_ref[...], preferred_element_type=jnp.float32)
    xp_scr[...] = (xp + b_ref[...]).reshape(tc // 2, 2 * B, G4)

    whh = whh_ref[...]

    def substep(xpt, h, c):
        hb = h.astype(jnp.bfloat16)
        gates = xpt + jnp.dot(hb, whh, preferred_element_type=jnp.float32)
        ig = jax.nn.sigmoid(gates[:, 0:GP])
        fg = jax.nn.sigmoid(gates[:, GP:2 * GP])
        gg = jnp.tanh(gates[:, 2 * GP:3 * GP])
        og = jax.nn.sigmoid(gates[:, 3 * GP:4 * GP])
        c_new = fg * c + ig * gg
        h_new = og * jnp.tanh(c_new)
        return h_new, c_new

    def step(j, carry):
        h, c = carry
        slab = xp_scr[j]
        h1, c1 = substep(slab[0:B], h, c)
        h2, c2 = substep(slab[B:2 * B], h1, c1)
        H_scr[k * (tc // 2) + j] = jnp.concatenate([h1, h2], axis=0)
        return (h2, c2)

    hF, cF = jax.lax.fori_loop(0, tc // 2, step, (h_scr[...], c_scr[...]),
                               unroll=32)
    h_scr[...] = hF
    c_scr[...] = cF

    @pl.when(k == nc - 1)
    def _final():
        sub_iota = jax.lax.broadcasted_iota(jnp.int32, (2 * B, 1), 0)
        rows = []
        for b in range(B):
            r = B * (len_ref[b] - 1) + b
            slab = H_scr[r // (2 * B)]
            onehot = (sub_iota == r % (2 * B)).astype(jnp.float32)
            rows.append(jnp.sum(slab * onehot, axis=0, keepdims=True))
        hT = jnp.concatenate(rows, axis=0)
        H = H_scr[...].reshape(T * B, GP)
        K2 = jnp.dot(H, wk_ref[...], preferred_element_type=jnp.float32) + bk_ref[...]
        V2 = jnp.dot(H, wv_ref[...], preferred_element_type=jnp.float32) + bv_ref[...]
        K4 = K2.reshape(T, B, D_HID)
        V4 = V2.reshape(T, B, D_HID)
        Q = jnp.dot(hT, wq_ref[...], preferred_element_type=jnp.float32) + bq_ref[...]
        L = jnp.sum(Q[None, :, :] * K4, axis=-1) * (1.0 / math.sqrt(D_HID))
        mf = maskt_ref[...]
        Lm = mf * L + (1.0 - mf) * (-1e9)
        mx = jnp.max(Lm, axis=0, keepdims=True)
        e = jnp.exp(Lm - mx)
        alpha = e / jnp.sum(e, axis=0, keepdims=True)
        alpha_ref[...] = alpha.T
        ctx = jnp.sum(alpha[:, :, None] * V4, axis=0)
        hT96 = hT[:, :D_HID]
        feats = jnp.concatenate([ctx, hT96], axis=-1)
        gl = jnp.dot(feats, wg_ref[...], preferred_element_type=jnp.float32) + bg_ref[...]
        gl_ref[...] = gl
        h1 = jnp.maximum(
            jnp.dot(feats, we1_ref[...], preferred_element_type=jnp.float32) + be1_ref[...],
            0.0)
        outs = jnp.dot(h1 * we2_ref[...], sseg_ref[...],
                       preferred_element_type=jnp.float32) + be2_ref[...]
        iota = jax.lax.broadcasted_iota(jnp.int32, (B, N_EXPERTS), 1)
        v1 = jnp.max(gl, axis=-1, keepdims=True)
        i1 = jnp.min(jnp.where(gl == v1, iota, N_EXPERTS), axis=-1, keepdims=True)
        m1 = iota == i1
        gl2 = jnp.where(m1, -1e30, gl)
        v2 = jnp.max(gl2, axis=-1, keepdims=True)
        i2 = jnp.min(jnp.where(gl2 == v2, iota, N_EXPERTS), axis=-1, keepdims=True)
        m2 = iota == i2
        d = jnp.exp(v2 - v1)
        p1 = 1.0 / (1.0 + d)
        p2 = d / (1.0 + d)
        s1 = jnp.sum(jnp.where(m1, outs, 0.0), axis=-1, keepdims=True)
        s2 = jnp.sum(jnp.where(m2, outs, 0.0), axis=-1, keepdims=True)
        yhat_ref[...] = p1 * s1 + p2 * s2


TC = 256


@jax.jit
def _run(x_tm, maskt, len2, wihT, whhT, b2,
         wqT, bq2, wkT, bk2, wvT, bv2,
         wgT, bg2, we1T, be1r, we2f, sseg, be2r):
    tc = TC
    nc = T // tc
    full = lambda shape: pl.BlockSpec(shape, lambda k: tuple(0 for _ in shape))
    out_shapes = (
        jax.ShapeDtypeStruct((B, 1), jnp.float32),
        jax.ShapeDtypeStruct((B, T), jnp.float32),
        jax.ShapeDtypeStruct((B, N_EXPERTS), jnp.float32),
    )
    return pl.pallas_call(
        functools.partial(_lstm_attn_moe_kernel, nc, tc),
        grid=(nc,),
        in_specs=[
            pl.BlockSpec((tc, B, D_IN), lambda k: (k, 0, 0)),
            full((T, B)),
            pl.BlockSpec(memory_space=pltpu.SMEM),
            full((D_IN, G4)),
            full((GP, G4)),
            full((1, G4)),
            full((GP, D_HID)), full((1, D_HID)),
            full((GP, D_HID)), full((1, D_HID)),
            full((GP, D_HID)), full((1, D_HID)),
            full((2 * D_HID, N_EXPERTS)), full((1, N_EXPERTS)),
            full((2 * D_HID, N_EXPERTS * D_HID)), full((1, N_EXPERTS * D_HID)),
            full((1, N_EXPERTS * D_HID)),
            full((N_EXPERTS * D_HID, N_EXPERTS)),
            full((1, N_EXPERTS)),
        ],
        out_specs=(full((B, 1)), full((B, T)), full((B, N_EXPERTS))),
        out_shape=out_shapes,
        scratch_shapes=[
            pltpu.VMEM((T // 2, 2 * B, GP), jnp.float32),
            pltpu.VMEM((B, GP), jnp.float32),
            pltpu.VMEM((B, GP), jnp.float32),
            pltpu.VMEM((tc // 2, 2 * B, G4), jnp.float32),
        ],
        compiler_params=pltpu.CompilerParams(
            dimension_semantics=("arbitrary",)),
    )(x_tm, maskt, len2, wihT, whhT, b2,
      wqT, bq2, wkT, bk2, wvT, bv2,
      wgT, bg2, we1T, be1r, we2f, sseg, be2r)


def _pad_rows(W):
    return jnp.pad(W, ((0, GP - D_HID), (0, 0)))


def kernel(x, lengths, mask, W_ih, W_hh, b_ih, b_hh, Wq, bq, Wk, bk, Wv, bv,
           Wg, bg, We1, be1, We2, be2):
    x_tm = jnp.swapaxes(x, 0, 1)
    maskt = mask.astype(jnp.float32).T
    len2 = lengths.astype(jnp.int32)
    wihT = _pad_gate_cols(W_ih.T)
    whhT = _pad_rows(_pad_gate_cols(W_hh.T)).astype(jnp.bfloat16)
    b2 = _pad_gate_cols((b_ih + b_hh).reshape(1, -1))
    we1T = We1.reshape(N_EXPERTS * D_HID, 2 * D_HID).T
    be1r = be1.reshape(1, N_EXPERTS * D_HID)
    we2f = We2.reshape(1, N_EXPERTS * D_HID)
    sseg = jnp.repeat(jnp.eye(N_EXPERTS, dtype=jnp.float32), D_HID, axis=0)
    be2r = be2.reshape(1, N_EXPERTS)
    return _run(x_tm, maskt, len2, wihT, whhT, b2,
                _pad_rows(Wq.T), bq.reshape(1, -1),
                _pad_rows(Wk.T), bk.reshape(1, -1),
                _pad_rows(Wv.T), bv.reshape(1, -1),
                Wg.T, bg.reshape(1, -1), we1T, be1r, we2f, sseg, be2r)


# speedup vs baseline: 1.0228x; 1.0156x over previous
import functools
import math

import jax
import jax.numpy as jnp
from jax.experimental import pallas as pl
from jax.experimental.pallas import tpu as pltpu

B = 4
T = 2048
D_IN = 768
D_HID = 96
N_EXPERTS = 8
GP = 128
G4 = 4 * GP


def _pad_gate_cols(Wt):
    z = jnp.zeros((Wt.shape[0], GP - D_HID), Wt.dtype)
    parts = []
    for j in range(4):
        parts.append(Wt[:, D_HID * j:D_HID * (j + 1)])
        parts.append(z)
    return jnp.concatenate(parts, axis=1)


def _lstm_attn_moe_kernel(nc, tc,
                          x_ref, maskf_ref, len_ref,
                          wih_ref, whh_ref, b_ref,
                          wq_ref, bq_ref, wk_ref, bk_ref, wv_ref, bv_ref,
                          wg_ref, bg_ref, we1_ref, be1_ref,
                          we2_ref, sseg_ref, be2_ref,
                          yhat_ref, alpha_ref, gl_ref,
                          H_scr, h_scr, c_scr, xp_scr):
    k = pl.program_id(0)

    @pl.when(k == 0)
    def _init():
        h_scr[...] = jnp.zeros_like(h_scr)
        c_scr[...] = jnp.zeros_like(c_scr)

    xb = x_ref[...]
    x2 = xb.reshape(B * tc, D_IN)
    xp = jnp.dot(x2, wih_ref[...], preferred_element_type=jnp.float32)
    xp_scr[...] = (xp + b_ref[...]).reshape(B, tc, G4)

    whh = whh_ref[...]

    def step(i, carry):
        h, c = carry
        t = k * tc + i
        xpt = xp_scr[:, i, :]
        gates = xpt + jnp.dot(h, whh, preferred_element_type=jnp.float32)
        ig = jax.nn.sigmoid(gates[:, 0:GP])
        fg = jax.nn.sigmoid(gates[:, GP:2 * GP])
        gg = jnp.tanh(gates[:, 2 * GP:3 * GP])
        og = jax.nn.sigmoid(gates[:, 3 * GP:4 * GP])
        c_new = fg * c + ig * gg
        h_new = og * jnp.tanh(c_new)
        H_scr[:, t, :] = h_new[:, :D_HID]
        return (h_new, c_new)

    hF, cF = jax.lax.fori_loop(0, tc, step, (h_scr[...], c_scr[...]),
                               unroll=32)
    h_scr[...] = hF
    c_scr[...] = cF

    @pl.when(k == nc - 1)
    def _final():
        rows = [H_scr[b, pl.ds(len_ref[b] - 1, 1), :] for b in range(B)]
        hT = jnp.concatenate(rows, axis=0)
        H = H_scr[...]
        H2 = H.reshape(B * T, D_HID)
        K2 = jnp.dot(H2, wk_ref[...], preferred_element_type=jnp.float32) + bk_ref[...]
        V2 = jnp.dot(H2, wv_ref[...], preferred_element_type=jnp.float32) + bv_ref[...]
        K3 = K2.reshape(B, T, D_HID)
        V3 = V2.reshape(B, T, D_HID)
        Q = jnp.dot(hT, wq_ref[...], preferred_element_type=jnp.float32) + bq_ref[...]
        L = jnp.sum(Q[:, None, :] * K3, axis=-1) * (1.0 / math.sqrt(D_HID))
        mf = maskf_ref[...]
        Lm = mf * L + (1.0 - mf) * (-1e9)
        mx = jnp.max(Lm, axis=-1, keepdims=True)
        e = jnp.exp(Lm - mx)
        alpha = e / jnp.sum(e, axis=-1, keepdims=True)
        alpha_ref[...] = alpha
        ctx = jnp.sum(alpha[:, :, None] * V3, axis=1)
        feats = jnp.concatenate([ctx, hT], axis=-1)
        gl = jnp.dot(feats, wg_ref[...], preferred_element_type=jnp.float32) + bg_ref[...]
        gl_ref[...] = gl
        h1 = jnp.maximum(
            jnp.dot(feats, we1_ref[...], preferred_element_type=jnp.float32) + be1_ref[...],
            0.0)
        outs = jnp.dot(h1 * we2_ref[...], sseg_ref[...],
                       preferred_element_type=jnp.float32) + be2_ref[...]
        iota = jax.lax.broadcasted_iota(jnp.int32, (B, N_EXPERTS), 1)
        v1 = jnp.max(gl, axis=-1, keepdims=True)
        i1 = jnp.min(jnp.where(gl == v1, iota, N_EXPERTS), axis=-1, keepdims=True)
        m1 = iota == i1
        gl2 = jnp.where(m1, -1e30, gl)
        v2 = jnp.max(gl2, axis=-1, keepdims=True)
        i2 = jnp.min(jnp.where(gl2 == v2, iota, N_EXPERTS), axis=-1, keepdims=True)
        m2 = iota == i2
        d = jnp.exp(v2 - v1)
        p1 = 1.0 / (1.0 + d)
        p2 = d / (1.0 + d)
        s1 = jnp.sum(jnp.where(m1, outs, 0.0), axis=-1, keepdims=True)
        s2 = jnp.sum(jnp.where(m2, outs, 0.0), axis=-1, keepdims=True)
        yhat_ref[...] = p1 * s1 + p2 * s2


TC = 256


@jax.jit
def _run(x, maskf, len2, wihT, whhT, b2,
         wqT, bq2, wkT, bk2, wvT, bv2,
         wgT, bg2, we1T, be1r, we2f, sseg, be2r):
    tc = TC
    nc = T // tc
    full = lambda shape: pl.BlockSpec(shape, lambda k: tuple(0 for _ in shape))
    out_shapes = (
        jax.ShapeDtypeStruct((B, 1), jnp.float32),
        jax.ShapeDtypeStruct((B, T), jnp.float32),
        jax.ShapeDtypeStruct((B, N_EXPERTS), jnp.float32),
    )
    return pl.pallas_call(
        functools.partial(_lstm_attn_moe_kernel, nc, tc),
        grid=(nc,),
        in_specs=[
            pl.BlockSpec((B, tc, D_IN), lambda k: (0, k, 0)),
            full((B, T)),
            pl.BlockSpec(memory_space=pltpu.SMEM),
            full((D_IN, G4)),
            full((GP, G4)),
            full((1, G4)),
            full((D_HID, D_HID)), full((1, D_HID)),
            full((D_HID, D_HID)), full((1, D_HID)),
            full((D_HID, D_HID)), full((1, D_HID)),
            full((2 * D_HID, N_EXPERTS)), full((1, N_EXPERTS)),
            full((2 * D_HID, N_EXPERTS * D_HID)), full((1, N_EXPERTS * D_HID)),
            full((1, N_EXPERTS * D_HID)),
            full((N_EXPERTS * D_HID, N_EXPERTS)),
            full((1, N_EXPERTS)),
        ],
        out_specs=(full((B, 1)), full((B, T)), full((B, N_EXPERTS))),
        out_shape=out_shapes,
        scratch_shapes=[
            pltpu.VMEM((B, T, D_HID), jnp.float32),
            pltpu.VMEM((B, GP), jnp.float32),
            pltpu.VMEM((B, GP), jnp.float32),
            pltpu.VMEM((B, tc, G4), jnp.float32),
        ],
        compiler_params=pltpu.CompilerParams(
            dimension_semantics=("arbitrary",)),
    )(x, maskf, len2, wihT, whhT, b2,
      wqT, bq2, wkT, bk2, wvT, bv2,
      wgT, bg2, we1T, be1r, we2f, sseg, be2r)


def kernel(x, lengths, mask, W_ih, W_hh, b_ih, b_hh, Wq, bq, Wk, bk, Wv, bv,
           Wg, bg, We1, be1, We2, be2):
    maskf = mask.astype(jnp.float32)
    len2 = lengths.astype(jnp.int32)
    wihT = _pad_gate_cols(W_ih.T)
    whhT = jnp.pad(_pad_gate_cols(W_hh.T), ((0, GP - D_HID), (0, 0)))
    b2 = _pad_gate_cols((b_ih + b_hh).reshape(1, -1))
    we1T = We1.reshape(N_EXPERTS * D_HID, 2 * D_HID).T
    be1r = be1.reshape(1, N_EXPERTS * D_HID)
    we2f = We2.reshape(1, N_EXPERTS * D_HID)
    sseg = jnp.repeat(jnp.eye(N_EXPERTS, dtype=jnp.float32), D_HID, axis=0)
    be2r = be2.reshape(1, N_EXPERTS)
    return _run(x, maskf, len2, wihT, whhT, b2,
                Wq.T, bq.reshape(1, -1), Wk.T, bk.reshape(1, -1),
                Wv.T, bv.reshape(1, -1),
                Wg.T, bg.reshape(1, -1), we1T, be1r, we2f, sseg, be2r)


# speedup vs baseline: 1.0239x; 1.0010x over previous
import functools
import math

import jax
import jax.numpy as jnp
from jax.experimental import pallas as pl
from jax.experimental.pallas import tpu as pltpu

B = 4
T = 2048
D_IN = 768
D_HID = 96
N_EXPERTS = 8
GP = 128
G4 = 4 * GP


def _pad_gate_cols(Wt):
    z = jnp.zeros((Wt.shape[0], GP - D_HID), Wt.dtype)
    parts = []
    for j in range(4):
        parts.append(Wt[:, D_HID * j:D_HID * (j + 1)])
        parts.append(z)
    return jnp.concatenate(parts, axis=1)


def _lstm_attn_moe_kernel(nc, tc,
                          x_ref, maskf_ref, len_ref,
                          wih_ref, whh_ref, b_ref,
                          wq_ref, bq_ref, wk_ref, bk_ref, wv_ref, bv_ref,
                          wg_ref, bg_ref, we1_ref, be1_ref,
                          we2_ref, sseg_ref, be2_ref,
                          yhat_ref, alpha_ref, gl_ref,
                          H_scr, h_scr, c_scr, xp_scr):
    k = pl.program_id(0)

    @pl.when(k == 0)
    def _init():
        h_scr[...] = jnp.zeros_like(h_scr)
        c_scr[...] = jnp.zeros_like(c_scr)

    xb = x_ref[...]
    x2 = xb.reshape(B * tc, D_IN)
    xp = jnp.dot(x2, wih_ref[...], preferred_element_type=jnp.float32)
    xp_scr[...] = (xp + b_ref[...]).reshape(B, tc, G4)

    whh = whh_ref[...]

    def step(i, carry):
        h, c = carry
        t = k * tc + i
        xpt = xp_scr[:, i, :]
        gates = xpt + jnp.dot(h, whh, preferred_element_type=jnp.float32)
        ig = jax.nn.sigmoid(gates[:, 0:GP])
        fg = jax.nn.sigmoid(gates[:, GP:2 * GP])
        gg = jnp.tanh(gates[:, 2 * GP:3 * GP])
        og = jax.nn.sigmoid(gates[:, 3 * GP:4 * GP])
        c_new = fg * c + ig * gg
        h_new = og * jnp.tanh(c_new)
        H_scr[:, t, :] = h_new[:, :D_HID]
        return (h_new, c_new)

    hF, cF = jax.lax.fori_loop(0, tc, step, (h_scr[...], c_scr[...]),
                               unroll=32)
    h_scr[...] = hF
    c_scr[...] = cF

    @pl.when(k == nc - 1)
    def _final():
        rows = [H_scr[b, pl.ds(len_ref[b] - 1, 1), :] for b in range(B)]
        hT = jnp.concatenate(rows, axis=0)
        H = H_scr[...]
        H2 = H.reshape(B * T, D_HID)
        K2 = jnp.dot(H2, wk_ref[...], preferred_element_type=jnp.float32) + bk_ref[...]
        V2 = jnp.dot(H2, wv_ref[...], preferred_element_type=jnp.float32) + bv_ref[...]
        K3 = K2.reshape(B, T, D_HID)
        V3 = V2.reshape(B, T, D_HID)
        Q = jnp.dot(hT, wq_ref[...], preferred_element_type=jnp.float32) + bq_ref[...]
        L = jnp.sum(Q[:, None, :] * K3, axis=-1) * (1.0 / math.sqrt(D_HID))
        mf = maskf_ref[...]
        Lm = mf * L + (1.0 - mf) * (-1e9)
        mx = jnp.max(Lm, axis=-1, keepdims=True)
        e = jnp.exp(Lm - mx)
        alpha = e / jnp.sum(e, axis=-1, keepdims=True)
        alpha_ref[...] = alpha
        ctx = jnp.sum(alpha[:, :, None] * V3, axis=1)
        feats = jnp.concatenate([ctx, hT], axis=-1)
        gl = jnp.dot(feats, wg_ref[...], preferred_element_type=jnp.float32) + bg_ref[...]
        gl_ref[...] = gl
        h1 = jnp.maximum(
            jnp.dot(feats, we1_ref[...], preferred_element_type=jnp.float32) + be1_ref[...],
            0.0)
        outs = jnp.dot(h1 * we2_ref[...], sseg_ref[...],
                       preferred_element_type=jnp.float32) + be2_ref[...]
        iota = jax.lax.broadcasted_iota(jnp.int32, (B, N_EXPERTS), 1)
        v1 = jnp.max(gl, axis=-1, keepdims=True)
        i1 = jnp.min(jnp.where(gl == v1, iota, N_EXPERTS), axis=-1, keepdims=True)
        m1 = iota == i1
        gl2 = jnp.where(m1, -1e30, gl)
        v2 = jnp.max(gl2, axis=-1, keepdims=True)
        i2 = jnp.min(jnp.where(gl2 == v2, iota, N_EXPERTS), axis=-1, keepdims=True)
        m2 = iota == i2
        d = jnp.exp(v2 - v1)
        p1 = 1.0 / (1.0 + d)
        p2 = d / (1.0 + d)
        s1 = jnp.sum(jnp.where(m1, outs, 0.0), axis=-1, keepdims=True)
        s2 = jnp.sum(jnp.where(m2, outs, 0.0), axis=-1, keepdims=True)
        yhat_ref[...] = p1 * s1 + p2 * s2


TC = 512


@jax.jit
def _run(x, maskf, len2, wihT, whhT, b2,
         wqT, bq2, wkT, bk2, wvT, bv2,
         wgT, bg2, we1T, be1r, we2f, sseg, be2r):
    tc = TC
    nc = T // tc
    full = lambda shape: pl.BlockSpec(shape, lambda k: tuple(0 for _ in shape))
    out_shapes = (
        jax.ShapeDtypeStruct((B, 1), jnp.float32),
        jax.ShapeDtypeStruct((B, T), jnp.float32),
        jax.ShapeDtypeStruct((B, N_EXPERTS), jnp.float32),
    )
    return pl.pallas_call(
        functools.partial(_lstm_attn_moe_kernel, nc, tc),
        grid=(nc,),
        in_specs=[
            pl.BlockSpec((B, tc, D_IN), lambda k: (0, k, 0)),
            full((B, T)),
            pl.BlockSpec(memory_space=pltpu.SMEM),
            full((D_IN, G4)),
            full((GP, G4)),
            full((1, G4)),
            full((D_HID, D_HID)), full((1, D_HID)),
            full((D_HID, D_HID)), full((1, D_HID)),
            full((D_HID, D_HID)), full((1, D_HID)),
            full((2 * D_HID, N_EXPERTS)), full((1, N_EXPERTS)),
            full((2 * D_HID, N_EXPERTS * D_HID)), full((1, N_EXPERTS * D_HID)),
            full((1, N_EXPERTS * D_HID)),
            full((N_EXPERTS * D_HID, N_EXPERTS)),
            full((1, N_EXPERTS)),
        ],
        out_specs=(full((B, 1)), full((B, T)), full((B, N_EXPERTS))),
        out_shape=out_shapes,
        scratch_shapes=[
            pltpu.VMEM((B, T, D_HID), jnp.float32),
            pltpu.VMEM((B, GP), jnp.float32),
            pltpu.VMEM((B, GP), jnp.float32),
            pltpu.VMEM((B, tc, G4), jnp.float32),
        ],
        compiler_params=pltpu.CompilerParams(
            dimension_semantics=("arbitrary",)),
    )(x, maskf, len2, wihT, whhT, b2,
      wqT, bq2, wkT, bk2, wvT, bv2,
      wgT, bg2, we1T, be1r, we2f, sseg, be2r)


def kernel(x, lengths, mask, W_ih, W_hh, b_ih, b_hh, Wq, bq, Wk, bk, Wv, bv,
           Wg, bg, We1, be1, We2, be2):
    maskf = mask.astype(jnp.float32)
    len2 = lengths.astype(jnp.int32)
    wihT = _pad_gate_cols(W_ih.T)
    whhT = jnp.pad(_pad_gate_cols(W_hh.T), ((0, GP - D_HID), (0, 0)))
    b2 = _pad_gate_cols((b_ih + b_hh).reshape(1, -1))
    we1T = We1.reshape(N_EXPERTS * D_HID, 2 * D_HID).T
    be1r = be1.reshape(1, N_EXPERTS * D_HID)
    we2f = We2.reshape(1, N_EXPERTS * D_HID)
    sseg = jnp.repeat(jnp.eye(N_EXPERTS, dtype=jnp.float32), D_HID, axis=0)
    be2r = be2.reshape(1, N_EXPERTS)
    return _run(x, maskf, len2, wihT, whhT, b2,
                Wq.T, bq.reshape(1, -1), Wk.T, bk.reshape(1, -1),
                Wv.T, bv.reshape(1, -1),
                Wg.T, bg.reshape(1, -1), we1T, be1r, we2f, sseg, be2r)
